# Initial kernel scaffold; baseline (speedup 1.0000x reference)
import functools

import jax
import jax.numpy as jnp
from jax import lax
from jax.experimental import pallas as pl
from jax.experimental.pallas import tpu as pltpu
from jax.experimental.pallas import tpu_sc as plsc

N = 10000
B = 8
F = 20000
E = 6 * F
NC = 2
NS = 16
NW = NC * NS
ROWS_PER_TILE = 640
NPAD = NS * ROWS_PER_TILE
KCH = 30
CHW = 128
EPT = KCH * CHW
EPAD = NW * EPT
BIG = 40.0


def _sc_body(pred16_hbm, rows_hbm, cols_hbm, out_hbm,
             vbuf, gbuf, ridx, cidx, pc16_s, acc_s, sem):
    c = lax.axis_index("c")
    s = lax.axis_index("s")
    wid = c * NS + s
    r0 = s * ROWS_PER_TILE

    pltpu.sync_copy(pred16_hbm.at[pl.ds(r0, ROWS_PER_TILE)], vbuf)

    def sig_step(i, _):
        v = vbuf[i]
        vbuf[i] = 1.0 / (1.0 + jnp.exp(-v))
        return 0

    lax.fori_loop(0, ROWS_PER_TILE, sig_step, 0)
    pltpu.sync_copy(vbuf, pc16_s.at[pl.ds(r0, ROWS_PER_TILE)])

    def zero_step(i, _):
        vbuf[i] = jnp.zeros((16,), jnp.float32)
        return 0

    lax.fori_loop(0, ROWS_PER_TILE, zero_step, 0)
    pltpu.sync_copy(vbuf, acc_s.at[pl.ds(r0, ROWS_PER_TILE)])
    plsc.subcore_barrier()

    pltpu.sync_copy(rows_hbm.at[wid], ridx)
    pltpu.sync_copy(cols_hbm.at[wid], cidx)

    def edge_step(j, _):
        pltpu.async_copy(pc16_s.at[cidx.at[j]], gbuf.at[j], sem).wait()
        pltpu.sync_copy(gbuf.at[j], acc_s.at[ridx.at[j]], add=True)
        return 0

    lax.fori_loop(0, KCH, edge_step, 0)
    plsc.subcore_barrier()

    pltpu.sync_copy(acc_s.at[pl.ds(r0, ROWS_PER_TILE)],
                    out_hbm.at[c, pl.ds(r0, ROWS_PER_TILE)])


def _reduce_body(part_ref, pred16_ref, out_ref):
    pc16 = jax.nn.sigmoid(pred16_ref[...])
    t = part_ref[0] + part_ref[1] + pc16
    mask = lax.broadcasted_iota(jnp.int32, (NPAD, B), 0) < N
    s8 = jnp.where(mask, t[:, 0:B], 0.0)
    nc8 = jnp.where(mask, t[:, B:2 * B] - t[:, 0:B], 0.0)
    pc8 = jnp.where(mask, pc16[:, 0:B], 0.0)
    pcn8 = jnp.where(mask, 1.0 - pc16[:, 0:B], 0.0)

    m_c = jnp.max(s8, axis=0, keepdims=True)
    m_nc = jnp.max(nc8, axis=0, keepdims=True)
    sum_c = jnp.sum(s8, axis=0, keepdims=True)
    sum_nc = jnp.sum(nc8, axis=0, keepdims=True)
    iso = (jnp.sum(jnp.abs(pc8 - s8 / (m_c + 1e-6)), axis=0, keepdims=True)
           + jnp.sum(jnp.abs(pcn8 - nc8 / (m_nc + 1e-6)), axis=0,
                     keepdims=True))
    norm = sum_c + sum_nc + 0.001
    out_ref[0, 0] = jnp.mean(jnp.log1p(iso / norm))


@jax.jit
def kernel(pred, faces):
    pred16 = jnp.concatenate(
        [pred.T, jnp.full((N, B), BIG, jnp.float32)], axis=1)
    pred16 = jnp.concatenate(
        [pred16, jnp.full((NPAD - N, 16), -BIG, jnp.float32)], axis=0)

    f0, f1, f2 = faces[:, 0], faces[:, 1], faces[:, 2]
    rows = jnp.concatenate([f0, f1, f2, f1, f2, f0])
    cols = jnp.concatenate([f1, f2, f0, f0, f1, f2])
    pad = jnp.full((EPAD - E,), N, jnp.int32)
    rows = jnp.concatenate([rows, pad]).reshape(NW, KCH, CHW)
    cols = jnp.concatenate([cols, pad]).reshape(NW, KCH, CHW)

    mesh = plsc.VectorSubcoreMesh(
        core_axis_name="c", subcore_axis_name="s",
        num_cores=NC, num_subcores=NS)
    part = pl.kernel(
        _sc_body,
        out_type=jax.ShapeDtypeStruct((NC, NPAD, 16), jnp.float32),
        mesh=mesh,
        scratch_types=[
            pltpu.VMEM((ROWS_PER_TILE, 16), jnp.float32),
            pltpu.VMEM((KCH, CHW, 16), jnp.float32),
            pltpu.VMEM((KCH, CHW), jnp.int32),
            pltpu.VMEM((KCH, CHW), jnp.int32),
            pltpu.VMEM_SHARED((NPAD, 16), jnp.float32),
            pltpu.VMEM_SHARED((NPAD, 16), jnp.float32),
            pltpu.SemaphoreType.DMA,
        ],
    )(pred16, rows, cols)

    loss = pl.pallas_call(
        _reduce_body,
        out_shape=jax.ShapeDtypeStruct((1, 1), jnp.float32),
    )(part, pred16)
    return loss[0, 0]


# speedup vs baseline: 10.2115x; 10.2115x over previous
import functools

import jax
import jax.numpy as jnp
from jax import lax
from jax.experimental import pallas as pl
from jax.experimental.pallas import tpu as pltpu
from jax.experimental.pallas import tpu_sc as plsc

N = 10000
B = 8
F = 20000
E = 6 * F
NC = 2
NS = 16
NW = NC * NS
ROWS_PER_TILE = 640
NPAD = NS * ROWS_PER_TILE
KCH = 30
CHW = 128
EPT = KCH * CHW
EPAD = NW * EPT
BIG = 40.0


def _sc_body(pred16_hbm, rows_hbm, cols_hbm, out_hbm,
             vbuf, gbuf, ridx, cidx, pc16_s, acc_s, sem):
    c = lax.axis_index("c")
    s = lax.axis_index("s")
    wid = c * NS + s
    r0 = s * ROWS_PER_TILE

    pltpu.sync_copy(pred16_hbm.at[pl.ds(r0, ROWS_PER_TILE)], vbuf)

    def sig_step(i, _):
        v = vbuf[i]
        vbuf[i] = 1.0 / (1.0 + jnp.exp(-v))
        return 0

    lax.fori_loop(0, ROWS_PER_TILE, sig_step, 0)
    pltpu.sync_copy(vbuf, pc16_s.at[pl.ds(r0, ROWS_PER_TILE)])

    def zero_step(i, _):
        vbuf[i] = jnp.zeros((16,), jnp.float32)
        return 0

    lax.fori_loop(0, ROWS_PER_TILE, zero_step, 0)
    pltpu.sync_copy(vbuf, acc_s.at[pl.ds(r0, ROWS_PER_TILE)])
    plsc.subcore_barrier()

    pltpu.sync_copy(rows_hbm.at[wid], ridx)
    pltpu.sync_copy(cols_hbm.at[wid], cidx)

    def edge_step(j, _):
        pltpu.async_copy(pc16_s.at[cidx.at[j]], gbuf.at[j], sem).wait()
        pltpu.sync_copy(gbuf.at[j], acc_s.at[ridx.at[j]], add=True)
        return 0

    lax.fori_loop(0, KCH, edge_step, 0)
    plsc.subcore_barrier()

    pltpu.sync_copy(acc_s.at[pl.ds(r0, ROWS_PER_TILE)],
                    out_hbm.at[c, pl.ds(r0, ROWS_PER_TILE)])


def _reduce_body(part_ref, pred16_ref, out_ref):
    pc16 = jax.nn.sigmoid(pred16_ref[...])
    t = part_ref[0] + part_ref[1] + pc16
    mask = lax.broadcasted_iota(jnp.int32, (NPAD, B), 0) < N
    s8 = jnp.where(mask, t[:, 0:B], 0.0)
    nc8 = jnp.where(mask, t[:, B:2 * B] - t[:, 0:B], 0.0)
    pc8 = jnp.where(mask, pc16[:, 0:B], 0.0)
    pcn8 = jnp.where(mask, 1.0 - pc16[:, 0:B], 0.0)

    m_c = jnp.max(s8, axis=0, keepdims=True)
    m_nc = jnp.max(nc8, axis=0, keepdims=True)
    sum_c = jnp.sum(s8, axis=0, keepdims=True)
    sum_nc = jnp.sum(nc8, axis=0, keepdims=True)
    iso = (jnp.sum(jnp.abs(pc8 - s8 / (m_c + 1e-6)), axis=0, keepdims=True)
           + jnp.sum(jnp.abs(pcn8 - nc8 / (m_nc + 1e-6)), axis=0,
                     keepdims=True))
    norm = sum_c + sum_nc + 0.001
    out_ref[...] = jnp.mean(jnp.log1p(iso / norm)).reshape(1, 1)


@jax.jit
def kernel(pred, faces):
    pred16 = jnp.concatenate(
        [pred.T, jnp.full((N, B), BIG, jnp.float32)], axis=1)
    pred16 = jnp.concatenate(
        [pred16, jnp.full((NPAD - N, 16), -BIG, jnp.float32)], axis=0)

    f0, f1, f2 = faces[:, 0], faces[:, 1], faces[:, 2]
    rows = jnp.concatenate([f0, f1, f2, f1, f2, f0])
    cols = jnp.concatenate([f1, f2, f0, f0, f1, f2])
    pad = jnp.full((EPAD - E,), N, jnp.int32)
    rows = jnp.concatenate([rows, pad]).reshape(NW, KCH, CHW)
    cols = jnp.concatenate([cols, pad]).reshape(NW, KCH, CHW)

    mesh = plsc.VectorSubcoreMesh(
        core_axis_name="c", subcore_axis_name="s",
        num_cores=NC, num_subcores=NS)
    part = pl.kernel(
        _sc_body,
        out_type=jax.ShapeDtypeStruct((NC, NPAD, 16), jnp.float32),
        mesh=mesh,
        scratch_types=[
            pltpu.VMEM((ROWS_PER_TILE, 16), jnp.float32),
            pltpu.VMEM((KCH, CHW, 16), jnp.float32),
            pltpu.VMEM((KCH, CHW), jnp.int32),
            pltpu.VMEM((KCH, CHW), jnp.int32),
            pltpu.VMEM_SHARED((NPAD, 16), jnp.float32),
            pltpu.VMEM_SHARED((NPAD, 16), jnp.float32),
            pltpu.SemaphoreType.DMA,
        ],
        compiler_params=pltpu.CompilerParams(use_tc_tiling_on_sc=False),
    )(pred16, rows, cols)

    loss = pl.pallas_call(
        _reduce_body,
        out_shape=jax.ShapeDtypeStruct((1, 1), jnp.float32),
    )(part, pred16)
    return loss[0, 0]


# speedup vs baseline: 11.6101x; 1.1370x over previous
import functools

import jax
import jax.numpy as jnp
from jax import lax
from jax.experimental import pallas as pl
from jax.experimental.pallas import tpu as pltpu
from jax.experimental.pallas import tpu_sc as plsc

N = 10000
B = 8
F = 20000
NC = 2
NS = 16
NW = NC * NS
RPT = 640
NPAD = NS * RPT
FPT = 640
FPAD = NW * FPT
KCH = 30
CHW = 128
EPT = KCH * CHW
BIG = 40.0
DIRS = ((0, 1), (1, 2), (2, 0), (1, 0), (2, 1), (0, 2))

NR = NPAD // 8
NVALID = N // 8


def _sc_body(pred_hbm, faces_hbm, out_hbm, pc_hbm,
             predt, vbuf, fbuf, gbuf, ridx, cidx, pc16_s, acc_s,
             sem0, sem1):
    c = lax.axis_index("c")
    s = lax.axis_index("s")
    wid = c * NS + s
    r0 = s * RPT
    lane = lax.iota(jnp.int32, 16)
    lane_lo = lane < 8

    for b in range(B):
        pltpu.sync_copy(pred_hbm.at[b, pl.ds(r0, RPT)],
                        predt.at[pl.ds(b * RPT, RPT)])

    def sig_step(i, _):
        v = plsc.load_gather(predt, [jnp.minimum(lane, 7) * RPT + i])
        sig = 1.0 / (1.0 + jnp.exp(-v))
        row = jnp.where(lane_lo, sig, 1.0)
        row = jnp.where(r0 + i < N, row, 0.0)
        vbuf[i] = row
        return 0

    lax.fori_loop(0, RPT, sig_step, 0)
    pltpu.sync_copy(vbuf, pc16_s.at[pl.ds(r0, RPT)])

    @pl.when(c == 0)
    def _():
        pltpu.sync_copy(vbuf, pc_hbm.at[pl.ds(r0, RPT)])

    def zero_step(i, _):
        z = jnp.zeros((16,), jnp.float32)
        for u in range(8):
            vbuf[i * 8 + u] = z
        return 0

    lax.fori_loop(0, RPT // 8, zero_step, 0)
    pltpu.sync_copy(vbuf, acc_s.at[pl.ds(r0, RPT)])

    pltpu.sync_copy(faces_hbm.at[wid], fbuf)

    def bld_step(v, _):
        base3 = (v * 16 + lane) * 3
        g = [plsc.load_gather(fbuf, [base3 + k]) for k in range(3)]
        row = v // 8
        co = (v % 8) * 16
        for d, (rc, cc) in enumerate(DIRS):
            ridx[5 * d + row, pl.ds(co, 16)] = g[rc]
            cidx[5 * d + row, pl.ds(co, 16)] = g[cc]
        return 0

    lax.fori_loop(0, FPT // 16, bld_step, 0)
    plsc.subcore_barrier()

    pltpu.async_copy(pc16_s.at[cidx.at[0]], gbuf.at[0], sem0)

    def edge_step(h, _):
        j = h * 2
        pltpu.async_copy(pc16_s.at[cidx.at[j + 1]], gbuf.at[j + 1], sem1)
        pltpu.make_async_copy(pc16_s.at[cidx.at[j]], gbuf.at[j], sem0).wait()
        pltpu.sync_copy(gbuf.at[j], acc_s.at[ridx.at[j]], add=True)

        @pl.when(j + 2 < KCH)
        def _():
            pltpu.async_copy(pc16_s.at[cidx.at[j + 2]], gbuf.at[j + 2], sem0)

        pltpu.make_async_copy(
            pc16_s.at[cidx.at[j + 1]], gbuf.at[j + 1], sem1).wait()
        pltpu.sync_copy(gbuf.at[j + 1], acc_s.at[ridx.at[j + 1]], add=True)
        return 0

    lax.fori_loop(0, KCH // 2, edge_step, 0)
    plsc.subcore_barrier()

    pltpu.sync_copy(acc_s.at[pl.ds(r0, RPT)],
                    out_hbm.at[c, pl.ds(r0, RPT)])


def _reduce_body(part_ref, pc_ref, out_ref):
    pc = pc_ref[...]
    t = part_ref[0] + part_ref[1] + pc
    lane16 = lax.broadcasted_iota(jnp.int32, (NR, 128), 1) % 16
    rowv = lax.broadcasted_iota(jnp.int32, (NR, 128), 0) < NVALID
    mask = jnp.logical_and(lane16 < 8, rowv)

    trot = jnp.concatenate([t[:, 8:], t[:, :8]], axis=1)
    s8 = jnp.where(mask, t, 0.0)
    nc8 = jnp.where(mask, trot - t, 0.0)
    pc8 = jnp.where(mask, pc, 0.0)
    pcn8 = jnp.where(mask, 1.0 - pc, 0.0)

    def fold(x):
        m = x[:, 0:16]
        for k in range(1, 8):
            m = jnp.maximum(m, x[:, 16 * k:16 * (k + 1)])
        return m

    def foldsum(x):
        m = x[:, 0:16]
        for k in range(1, 8):
            m = m + x[:, 16 * k:16 * (k + 1)]
        return m

    def widen(x16):
        return jnp.concatenate([x16] * 8, axis=1)

    m_c = widen(fold(jnp.max(s8, axis=0, keepdims=True)))
    m_nc = widen(fold(jnp.max(nc8, axis=0, keepdims=True)))
    sum_c = foldsum(jnp.sum(s8, axis=0, keepdims=True))
    sum_nc = foldsum(jnp.sum(nc8, axis=0, keepdims=True))

    iso128 = (jnp.sum(jnp.abs(pc8 - s8 / (m_c + 1e-6)), axis=0, keepdims=True)
              + jnp.sum(jnp.abs(pcn8 - nc8 / (m_nc + 1e-6)), axis=0,
                        keepdims=True))
    iso = foldsum(iso128)
    norm = sum_c + sum_nc + 0.001
    pen = jnp.log1p(iso / norm)
    b16 = lax.broadcasted_iota(jnp.int32, (1, 16), 1) < B
    out_ref[...] = (jnp.sum(jnp.where(b16, pen, 0.0)) / B).reshape(1, 1)


@jax.jit
def kernel(pred, faces):
    faces_pad = jnp.concatenate(
        [faces, jnp.full((FPAD - F, 3), N, jnp.int32)]).reshape(NW, FPT * 3)

    mesh = plsc.VectorSubcoreMesh(
        core_axis_name="c", subcore_axis_name="s",
        num_cores=NC, num_subcores=NS)
    part, pc16 = pl.kernel(
        _sc_body,
        out_type=(jax.ShapeDtypeStruct((NC, NPAD, 16), jnp.float32),
                  jax.ShapeDtypeStruct((NPAD, 16), jnp.float32)),
        mesh=mesh,
        scratch_types=[
            pltpu.VMEM((B * RPT,), jnp.float32),
            pltpu.VMEM((RPT, 16), jnp.float32),
            pltpu.VMEM((FPT * 3,), jnp.int32),
            pltpu.VMEM((KCH, CHW, 16), jnp.float32),
            pltpu.VMEM((KCH, CHW), jnp.int32),
            pltpu.VMEM((KCH, CHW), jnp.int32),
            pltpu.VMEM_SHARED((NPAD, 16), jnp.float32),
            pltpu.VMEM_SHARED((NPAD, 16), jnp.float32),
            pltpu.SemaphoreType.DMA,
            pltpu.SemaphoreType.DMA,
        ],
        compiler_params=pltpu.CompilerParams(
            use_tc_tiling_on_sc=False, needs_layout_passes=False),
    )(pred, faces_pad)

    loss = pl.pallas_call(
        _reduce_body,
        out_shape=jax.ShapeDtypeStruct((1, 1), jnp.float32),
    )(part.reshape(NC, NR, 128), pc16.reshape(NR, 128))
    return loss[0, 0]
